# baseline (speedup 1.0000x reference)
import functools

import jax
import jax.numpy as jnp
from jax import lax
from jax.experimental import pallas as pl
from jax.experimental.pallas import tpu as pltpu
from jax.experimental.pallas import tpu_sc as plsc

SEGS = 128
DF = 128
N = 100000
NC, NS, L = 2, 16, 16
NW = NC * NS
RPW = N // NW
G = 8
NSLOT = 4
BLK = 120
NB = RPW // BLK
NGRP = BLK // G
TAIL = RPW - NB * BLK
NG = DF // L
BSTRIDE = 136
BCHUNK = 3136

_f32 = jnp.float32
_i32 = jnp.int32


def _conv_pass(xref, bref, conv, wid, psum_o, pmax_o, xbuf, bbuf, lsum,
               lmax, sem):
    zeros = jnp.zeros((L,), _f32)
    ninf = jnp.full((L,), -jnp.inf, _f32)

    def init_body(i, _):
        lsum[pl.ds(i * L, L)] = zeros
        lmax[pl.ds(i * L, L)] = ninf
        return 0
    lax.fori_loop(0, SEGS * DF // L, init_body, 0)

    row0 = wid * RPW
    loc = lax.rem(row0, 8)
    astart = pl.multiple_of(row0 - loc, 8)
    pltpu.sync_copy(bref.at[pl.ds(astart, BCHUNK)],
                    bbuf.at[pl.ds(0, BCHUNK)])

    def xsrc(b, nrows):
        return xref.at[pl.ds(pl.multiple_of((row0 + b * BLK) * DF, 8),
                             nrows * DF)]

    pltpu.make_async_copy(xsrc(0, BLK), xbuf.at[0, pl.ds(0, BLK * DF)],
                          sem).start()
    pltpu.make_async_copy(xsrc(1, BLK), xbuf.at[1, pl.ds(0, BLK * DF)],
                          sem).start()
    pltpu.make_async_copy(xsrc(2, BLK), xbuf.at[2, pl.ds(0, BLK * DF)],
                          sem).start()

    def merge_acc(seg, accs):
        off = seg * DF
        for j in range(NG):
            lsum[pl.ds(off + L * j, L)] = (
                lsum[pl.ds(off + L * j, L)] + accs[j])
            lmax[pl.ds(off + L * j, L)] = jnp.maximum(
                lmax[pl.ds(off + L * j, L)], accs[NG + j])

    def rmw_row(seg, xr):
        off = seg * DF
        for j in range(NG):
            lsum[pl.ds(off + L * j, L)] = (
                lsum[pl.ds(off + L * j, L)] + xr[j])
            lmax[pl.ds(off + L * j, L)] = jnp.maximum(
                lmax[pl.ds(off + L * j, L)], xr[j])

    neutral = (zeros,) * NG + (ninf,) * NG

    def tree_group(slot, gbase):
        g_s = []
        g_m = []
        for j in range(NG):
            r = [xbuf[slot, pl.ds(gbase + k * DF + L * j, L)]
                 for k in range(G)]
            a0, a1 = r[0] + r[1], r[2] + r[3]
            a2, a3 = r[4] + r[5], r[6] + r[7]
            g_s.append((a0 + a1) + (a2 + a3))
            m0, m1 = jnp.maximum(r[0], r[1]), jnp.maximum(r[2], r[3])
            m2, m3 = jnp.maximum(r[4], r[5]), jnp.maximum(r[6], r[7])
            g_m.append(jnp.maximum(jnp.maximum(m0, m1),
                                   jnp.maximum(m2, m3)))
        return g_s, g_m

    def block_body(b, _):
        slot = lax.rem(b, NSLOT)
        pltpu.make_async_copy(xsrc(b, BLK), xbuf.at[slot, pl.ds(0, BLK * DF)],
                              sem).wait()

        @pl.when(b + 3 < NB)
        def _():
            pltpu.make_async_copy(
                xsrc(b + 3, BLK),
                xbuf.at[lax.rem(b + 3, NSLOT), pl.ds(0, BLK * DF)],
                sem).start()

        bbase = loc + b * BLK
        sA = bbuf[pl.ds(bbase, L)][0]
        sB = bbuf[pl.ds(bbase + BLK - L, L)][L - 1]

        @pl.when(sA == sB)
        def _():
            def fast_group(g, accs):
                g_s, g_m = tree_group(slot, g * (G * DF))
                return (tuple(accs[j] + g_s[j] for j in range(NG))
                        + tuple(jnp.maximum(accs[NG + j], g_m[j])
                                for j in range(NG)))

            accs = lax.fori_loop(0, NGRP, fast_group, neutral)
            merge_acc(sA, accs)

        @pl.when(sA != sB)
        def _():
            def group_body(g, carry):
                prev_seg, accs = carry
                segv = bbuf[pl.ds(bbase + g * G, L)]
                s0 = segv[0]
                s7 = segv[G - 1]
                uniform = s0 == s7
                cont = uniform & (s0 == prev_seg)

                @pl.when(jnp.logical_not(cont))
                def _():
                    merge_acc(prev_seg, accs)

                gbase = g * (G * DF)
                g_s, g_m = tree_group(slot, gbase)

                @pl.when(jnp.logical_not(uniform))
                def _():
                    for k in range(G):
                        sk = segv[k]
                        xr = [xbuf[slot, pl.ds(gbase + k * DF + L * j, L)]
                              for j in range(NG)]
                        rmw_row(sk, xr)

                new_s = tuple(
                    jnp.where(cont, accs[j] + g_s[j],
                              jnp.where(uniform, g_s[j], zeros))
                    for j in range(NG))
                new_m = tuple(
                    jnp.where(cont, jnp.maximum(accs[NG + j], g_m[j]),
                              jnp.where(uniform, g_m[j], ninf))
                    for j in range(NG))
                return s7, new_s + new_m

            prev_seg, accs = lax.fori_loop(0, NGRP, group_body,
                                           (sA, neutral))
            merge_acc(prev_seg, accs)

        return 0

    lax.fori_loop(0, NB, block_body, 0)

    pltpu.sync_copy(xsrc(NB, TAIL), xbuf.at[0, pl.ds(0, TAIL * DF)])
    for k in range(TAIL):
        sk = bbuf[pl.ds(loc + NB * BLK + k, L)][0]
        xr = [xbuf[0, pl.ds(k * DF + L * j, L)] for j in range(NG)]
        rmw_row(sk, xr)

    wslot = conv * NW + wid
    pltpu.sync_copy(
        lsum, psum_o.at[pl.ds(pl.multiple_of(wslot * SEGS * DF, 8),
                              SEGS * DF)])
    pltpu.sync_copy(
        lmax, pmax_o.at[pl.ds(pl.multiple_of(wslot * SEGS * DF, 8),
                              SEGS * DF)])


@functools.partial(
    pl.kernel,
    out_type=[
        jax.ShapeDtypeStruct((2 * NW * SEGS * DF,), _f32),
        jax.ShapeDtypeStruct((2 * NW * SEGS * DF,), _f32),
    ],
    mesh=plsc.VectorSubcoreMesh(core_axis_name="c", subcore_axis_name="s"),
    scratch_types=[
        pltpu.VMEM((NSLOT, BLK * DF), _f32),
        pltpu.VMEM((BCHUNK + L,), _i32),
        pltpu.VMEM((SEGS * DF,), _f32),
        pltpu.VMEM((SEGS * DF,), _f32),
        pltpu.SemaphoreType.DMA,
    ],
)
def _phase1(x0, b0, x1, b1, psum_o, pmax_o,
            xbuf, bbuf, lsum, lmax, sem):
    wid = lax.axis_index("s") * NC + lax.axis_index("c")
    for conv, (xref, bref) in enumerate(((x0, b0), (x1, b1))):
        _conv_pass(xref, bref, conv, wid, psum_o, pmax_o, xbuf, bbuf,
                   lsum, lmax, sem)


def _phase2_body(psum_ref, pmax_ref, invc_ref, out_ref):
    s = jnp.sum(psum_ref[0], axis=0)
    mean = s * invc_ref[0]
    mx = jnp.max(pmax_ref[0], axis=0)
    out_ref[:, :DF] = mean
    out_ref[:, DF:] = mx


TCR = 2000
TCNB = N // TCR


def _tcconv_body(sb_ref, x_ref, bcol_ref, brow_ref, invc_ref, out_ref,
                 acc_s, acc_m):
    i = pl.program_id(0)
    x = x_ref[...]
    brow = brow_ref[0]
    iot = lax.broadcasted_iota(_i32, (SEGS, TCR), 0).astype(_f32)
    oh = (iot == brow).astype(_f32)
    psum = jax.lax.dot(oh, x, precision=jax.lax.Precision.HIGHEST,
                       preferred_element_type=_f32)

    @pl.when(i == 0)
    def _():
        acc_s[...] = psum
        acc_m[...] = jnp.full((SEGS, DF), -jnp.inf, _f32)

    @pl.when(i > 0)
    def _():
        acc_s[...] = acc_s[...] + psum

    bcol = bcol_ref[0]
    slo = sb_ref[0, i]
    shi = sb_ref[1, i]

    def seg_body(s, _):
        mask = bcol == s.astype(_f32)
        xm = jnp.where(mask, x, -jnp.inf)
        ms = jnp.max(xm, axis=0, keepdims=True)
        acc_m[pl.ds(s, 1), :] = jnp.maximum(acc_m[pl.ds(s, 1), :], ms)
        return 0

    lax.fori_loop(slo, shi + 1, seg_body, 0)

    @pl.when(i == TCNB - 1)
    def _():
        out_ref[:, :DF] = acc_s[...] * invc_ref[...]
        out_ref[:, DF:] = acc_m[...]


def kernel(x0, batch0, x1, batch1, x2, batch2):
    xs = [x.reshape(N * DF) for x in (x0, x1)]
    bs = [jnp.pad(b.astype(_i32), (0, 32)) for b in (batch0, batch1)]
    pts = jnp.arange(SEGS + 1, dtype=_i32)
    bnd = jnp.stack([
        jnp.searchsorted(b.astype(_i32), pts, side="left").astype(_i32)
        for b in (batch0, batch1, batch2)])
    counts = (bnd[:, 1:] - bnd[:, :-1]).astype(_f32)
    invc = (1.0 / jnp.maximum(counts, 1.0)).reshape(3, SEGS, 1)

    b2 = batch2.astype(_i32)
    sb = jnp.stack([b2[::TCR], b2[TCR - 1::TCR]])
    b2f = b2.astype(_f32)
    out2 = pl.pallas_call(
        _tcconv_body,
        grid=(TCNB,),
        in_specs=[
            pl.BlockSpec(memory_space=pltpu.SMEM),
            pl.BlockSpec((TCR, DF), lambda i: (i, 0)),
            pl.BlockSpec((1, TCR, 1), lambda i: (i, 0, 0)),
            pl.BlockSpec((1, 1, TCR), lambda i: (i, 0, 0)),
            pl.BlockSpec((SEGS, 1), lambda i: (0, 0)),
        ],
        out_specs=pl.BlockSpec((SEGS, 2 * DF), lambda i: (0, 0)),
        out_shape=jax.ShapeDtypeStruct((SEGS, 2 * DF), _f32),
        scratch_shapes=[
            pltpu.VMEM((SEGS, DF), _f32),
            pltpu.VMEM((SEGS, DF), _f32),
        ],
    )(sb, x2, b2f.reshape(TCNB, TCR, 1), b2f.reshape(TCNB, 1, TCR),
      invc[2])

    psum, pmax = _phase1(xs[0], bs[0], xs[1], bs[1])
    psum = psum.reshape(2, NW, SEGS, DF)
    pmax = pmax.reshape(2, NW, SEGS, DF)
    out01 = pl.pallas_call(
        _phase2_body,
        grid=(2,),
        in_specs=[
            pl.BlockSpec((1, NW, SEGS, DF), lambda i: (i, 0, 0, 0)),
            pl.BlockSpec((1, NW, SEGS, DF), lambda i: (i, 0, 0, 0)),
            pl.BlockSpec((1, SEGS, 1), lambda i: (i, 0, 0)),
        ],
        out_specs=pl.BlockSpec((SEGS, 2 * DF), lambda i: (0, i)),
        out_shape=jax.ShapeDtypeStruct((SEGS, 2 * 2 * DF), _f32),
    )(psum, pmax, invc[:2])
    return jnp.concatenate([out01, out2], axis=1)


# speedup vs baseline: 1.8937x; 1.8937x over previous
import functools

import jax
import jax.numpy as jnp
from jax import lax
from jax.experimental import pallas as pl
from jax.experimental.pallas import tpu as pltpu
from jax.experimental.pallas import tpu_sc as plsc

SEGS = 128
DF = 128
N = 100000
NC, NS, L = 2, 16, 16
NW = NC * NS
RPW = N // NW
G = 8
NSLOT = 4
BLK = 120
NB = RPW // BLK
NGRP = BLK // G
TAIL = RPW - NB * BLK
NG = DF // L
BSTRIDE = 136
BCHUNK = 3136

_f32 = jnp.float32
_i32 = jnp.int32


def _conv_pass(xref, bref, conv, wid, psum_o, pmax_o, xbuf, bbuf, lsum,
               lmax, sem):
    zeros = jnp.zeros((L,), _f32)
    ninf = jnp.full((L,), -jnp.inf, _f32)

    def init_body(i, _):
        lsum[pl.ds(i * L, L)] = zeros
        lmax[pl.ds(i * L, L)] = ninf
        return 0
    lax.fori_loop(0, SEGS * DF // L, init_body, 0)

    row0 = wid * RPW
    loc = lax.rem(row0, 8)
    astart = pl.multiple_of(row0 - loc, 8)
    pltpu.sync_copy(bref.at[pl.ds(astart, BCHUNK)],
                    bbuf.at[pl.ds(0, BCHUNK)])

    def xsrc(b, nrows):
        return xref.at[pl.ds(pl.multiple_of((row0 + b * BLK) * DF, 8),
                             nrows * DF)]

    pltpu.make_async_copy(xsrc(0, BLK), xbuf.at[0, pl.ds(0, BLK * DF)],
                          sem).start()
    pltpu.make_async_copy(xsrc(1, BLK), xbuf.at[1, pl.ds(0, BLK * DF)],
                          sem).start()
    pltpu.make_async_copy(xsrc(2, BLK), xbuf.at[2, pl.ds(0, BLK * DF)],
                          sem).start()

    def merge_acc(seg, accs):
        off = seg * DF
        for j in range(NG):
            lsum[pl.ds(off + L * j, L)] = (
                lsum[pl.ds(off + L * j, L)] + accs[j])
            lmax[pl.ds(off + L * j, L)] = jnp.maximum(
                lmax[pl.ds(off + L * j, L)], accs[NG + j])

    def rmw_row(seg, xr):
        off = seg * DF
        for j in range(NG):
            lsum[pl.ds(off + L * j, L)] = (
                lsum[pl.ds(off + L * j, L)] + xr[j])
            lmax[pl.ds(off + L * j, L)] = jnp.maximum(
                lmax[pl.ds(off + L * j, L)], xr[j])

    neutral = (zeros,) * NG + (ninf,) * NG

    def tree_group(slot, gbase):
        g_s = []
        g_m = []
        for j in range(NG):
            r = [xbuf[slot, pl.ds(gbase + k * DF + L * j, L)]
                 for k in range(G)]
            a0, a1 = r[0] + r[1], r[2] + r[3]
            a2, a3 = r[4] + r[5], r[6] + r[7]
            g_s.append((a0 + a1) + (a2 + a3))
            m0, m1 = jnp.maximum(r[0], r[1]), jnp.maximum(r[2], r[3])
            m2, m3 = jnp.maximum(r[4], r[5]), jnp.maximum(r[6], r[7])
            g_m.append(jnp.maximum(jnp.maximum(m0, m1),
                                   jnp.maximum(m2, m3)))
        return g_s, g_m

    def block_body(b, _):
        slot = lax.rem(b, NSLOT)
        pltpu.make_async_copy(xsrc(b, BLK), xbuf.at[slot, pl.ds(0, BLK * DF)],
                              sem).wait()

        @pl.when(b + 3 < NB)
        def _():
            pltpu.make_async_copy(
                xsrc(b + 3, BLK),
                xbuf.at[lax.rem(b + 3, NSLOT), pl.ds(0, BLK * DF)],
                sem).start()

        bbase = loc + b * BLK
        sA = bbuf[pl.ds(bbase, L)][0]
        sB = bbuf[pl.ds(bbase + BLK - L, L)][L - 1]

        @pl.when(sA == sB)
        def _():
            def fast_group(g, accs):
                g_s, g_m = tree_group(slot, g * (G * DF))
                return (tuple(accs[j] + g_s[j] for j in range(NG))
                        + tuple(jnp.maximum(accs[NG + j], g_m[j])
                                for j in range(NG)))

            accs = lax.fori_loop(0, NGRP, fast_group, neutral)
            merge_acc(sA, accs)

        @pl.when(sA != sB)
        def _():
            def group_body(g, carry):
                prev_seg, accs = carry
                segv = bbuf[pl.ds(bbase + g * G, L)]
                s0 = segv[0]
                s7 = segv[G - 1]
                uniform = s0 == s7
                cont = uniform & (s0 == prev_seg)

                @pl.when(jnp.logical_not(cont))
                def _():
                    merge_acc(prev_seg, accs)

                gbase = g * (G * DF)
                g_s, g_m = tree_group(slot, gbase)

                @pl.when(jnp.logical_not(uniform))
                def _():
                    for k in range(G):
                        sk = segv[k]
                        xr = [xbuf[slot, pl.ds(gbase + k * DF + L * j, L)]
                              for j in range(NG)]
                        rmw_row(sk, xr)

                new_s = tuple(
                    jnp.where(cont, accs[j] + g_s[j],
                              jnp.where(uniform, g_s[j], zeros))
                    for j in range(NG))
                new_m = tuple(
                    jnp.where(cont, jnp.maximum(accs[NG + j], g_m[j]),
                              jnp.where(uniform, g_m[j], ninf))
                    for j in range(NG))
                return s7, new_s + new_m

            prev_seg, accs = lax.fori_loop(0, NGRP, group_body,
                                           (sA, neutral))
            merge_acc(prev_seg, accs)

        return 0

    lax.fori_loop(0, NB, block_body, 0)

    pltpu.sync_copy(xsrc(NB, TAIL), xbuf.at[0, pl.ds(0, TAIL * DF)])
    for k in range(TAIL):
        sk = bbuf[pl.ds(loc + NB * BLK + k, L)][0]
        xr = [xbuf[0, pl.ds(k * DF + L * j, L)] for j in range(NG)]
        rmw_row(sk, xr)

    wslot = conv * NW + wid
    pltpu.sync_copy(
        lsum, psum_o.at[pl.ds(pl.multiple_of(wslot * SEGS * DF, 8),
                              SEGS * DF)])
    pltpu.sync_copy(
        lmax, pmax_o.at[pl.ds(pl.multiple_of(wslot * SEGS * DF, 8),
                              SEGS * DF)])


@functools.partial(
    pl.kernel,
    out_type=[
        jax.ShapeDtypeStruct((3 * NW * SEGS * DF,), _f32),
        jax.ShapeDtypeStruct((3 * NW * SEGS * DF,), _f32),
    ],
    mesh=plsc.VectorSubcoreMesh(core_axis_name="c", subcore_axis_name="s"),
    scratch_types=[
        pltpu.VMEM((NSLOT, BLK * DF), _f32),
        pltpu.VMEM((BCHUNK + L,), _i32),
        pltpu.VMEM((SEGS * DF,), _f32),
        pltpu.VMEM((SEGS * DF,), _f32),
        pltpu.SemaphoreType.DMA,
    ],
)
def _phase1(x0, b0, x1, b1, x2, b2, psum_o, pmax_o,
            xbuf, bbuf, lsum, lmax, sem):
    wid = lax.axis_index("s") * NC + lax.axis_index("c")
    for conv, (xref, bref) in enumerate(((x0, b0), (x1, b1), (x2, b2))):
        _conv_pass(xref, bref, conv, wid, psum_o, pmax_o, xbuf, bbuf,
                   lsum, lmax, sem)


def _phase2_body(psum_ref, pmax_ref, invc_ref, out_ref):
    s = jnp.sum(psum_ref[0], axis=0)
    mean = s * invc_ref[0]
    mx = jnp.max(pmax_ref[0], axis=0)
    out_ref[:, :DF] = mean
    out_ref[:, DF:] = mx


def kernel(x0, batch0, x1, batch1, x2, batch2):
    xs = [x.reshape(N * DF) for x in (x0, x1, x2)]
    bs = [jnp.pad(b.astype(_i32), (0, 32))
          for b in (batch0, batch1, batch2)]
    pts = jnp.arange(SEGS + 1, dtype=_i32)
    bnd = jnp.stack([
        jnp.searchsorted(b.astype(_i32), pts, side="left").astype(_i32)
        for b in (batch0, batch1, batch2)])
    psum, pmax = _phase1(xs[0], bs[0], xs[1], bs[1], xs[2], bs[2])
    psum = psum.reshape(3, NW, SEGS, DF)
    pmax = pmax.reshape(3, NW, SEGS, DF)
    counts = (bnd[:, 1:] - bnd[:, :-1]).astype(_f32)
    invc = (1.0 / jnp.maximum(counts, 1.0)).reshape(3, SEGS, 1)
    out = pl.pallas_call(
        _phase2_body,
        grid=(3,),
        in_specs=[
            pl.BlockSpec((1, NW, SEGS, DF), lambda i: (i, 0, 0, 0)),
            pl.BlockSpec((1, NW, SEGS, DF), lambda i: (i, 0, 0, 0)),
            pl.BlockSpec((1, SEGS, 1), lambda i: (i, 0, 0)),
        ],
        out_specs=pl.BlockSpec((SEGS, 2 * DF), lambda i: (0, i)),
        out_shape=jax.ShapeDtypeStruct((SEGS, 3 * 2 * DF), _f32),
    )(psum, pmax, invc)
    return out
